# baseline (speedup 1.0000x reference)
import functools

import jax
import jax.numpy as jnp
from jax import lax
from jax.experimental import pallas as pl
from jax.experimental.pallas import tpu as pltpu
from jax.experimental.pallas import tpu_sc as plsc


def _sc_geometry():
    try:
        info = plsc.get_sparse_core_info()
        return info.num_cores, info.num_subcores
    except Exception:
        return 2, 16


def _embed_lookup(x2d, table, pe):
    B, S = x2d.shape
    V, D = table.shape
    NC, NS = _sc_geometry()
    NW = NC * NS
    C = S // NW

    H = C // 2
    NSTEP = 2 * B

    mesh = plsc.VectorSubcoreMesh(core_axis_name="c", subcore_axis_name="s")

    @functools.partial(
        pl.kernel,
        mesh=mesh,
        out_type=jax.ShapeDtypeStruct((B, S, D), jnp.float32),
        scratch_types=[
            [pltpu.VMEM((H,), jnp.int32)] * 2,
            [pltpu.VMEM((H, D), jnp.float32)] * 2,
            pltpu.VMEM((C, D), jnp.float32),
            [pltpu.SemaphoreType.DMA] * 2,
            [pltpu.SemaphoreType.DMA] * 2,
        ],
    )
    def emb(x_hbm, table_hbm, pe_hbm, out_hbm, idx_v, rows_v, pe_v, gsem, wsem):
        wid = lax.axis_index("s") * NC + lax.axis_index("c")
        base = wid * C
        pltpu.sync_copy(pe_hbm.at[pl.ds(base, C)], pe_v)
        nj = D // 16

        def start_gather(s):
            b, h = divmod(s, 2)
            p = s % 2
            pltpu.sync_copy(x_hbm.at[b, pl.ds(base + h * H, H)], idx_v[p])
            return pltpu.async_copy(table_hbm.at[idx_v[p]], rows_v[p], gsem[p])

        def start_write(s):
            b, h = divmod(s, 2)
            p = s % 2
            return pltpu.async_copy(
                rows_v[p], out_hbm.at[b, pl.ds(base + h * H, H)], wsem[p])

        gdesc = {0: start_gather(0)}
        wdesc = {}
        for s in range(NSTEP):
            p = s % 2
            b, h = divmod(s, 2)
            gdesc.pop(s).wait()
            if s + 1 < NSTEP:
                if s >= 1:
                    wdesc.pop(s - 1).wait()
                gdesc[s + 1] = start_gather(s + 1)

            def add_pe_row(r, carry, _h=h, _p=p):
                for j in range(nj):
                    plsc.addupdate(rows_v[_p].at[r, pl.ds(j * 16, 16)],
                                   pe_v[_h * H + r, pl.ds(j * 16, 16)])
                return carry

            lax.fori_loop(0, H, add_pe_row, 0)
            wdesc[s] = start_write(s)
        for s in sorted(wdesc):
            wdesc.pop(s).wait()

    return emb(x2d, table, pe)


def kernel(x, table, pe):
    return _embed_lookup(x.astype(jnp.int32), table, pe.astype(jnp.float32))


# speedup vs baseline: 1.0022x; 1.0022x over previous
import functools

import jax
import jax.numpy as jnp
from jax import lax
from jax.experimental import pallas as pl
from jax.experimental.pallas import tpu as pltpu
from jax.experimental.pallas import tpu_sc as plsc


def _sc_geometry():
    try:
        info = plsc.get_sparse_core_info()
        return info.num_cores, info.num_subcores
    except Exception:
        return 2, 16


def _embed_lookup(x2d, table, pe):
    B, S = x2d.shape
    V, D = table.shape
    NC, NS = _sc_geometry()
    NW = NC * NS
    C = S // NW

    mesh = plsc.VectorSubcoreMesh(core_axis_name="c", subcore_axis_name="s")

    @functools.partial(
        pl.kernel,
        mesh=mesh,
        out_type=jax.ShapeDtypeStruct((B, S, D), jnp.float32),
        scratch_types=[
            [pltpu.VMEM((C // 2,), jnp.int32)] * 3,
            [pltpu.VMEM((C // 2, D), jnp.float32)] * 3,
            pltpu.VMEM((C, D), jnp.float32),
            [pltpu.SemaphoreType.DMA] * 3,
            [pltpu.SemaphoreType.DMA] * 3,
        ],
    )
    def emb(x_hbm, table_hbm, pe_hbm, out_hbm, idx_v, rows_v, pe_v, gsem, wsem):
        wid = lax.axis_index("s") * NC + lax.axis_index("c")
        base = wid * C
        H = C // 2
        NSTEP = 2 * B
        nj = D // 16
        pltpu.sync_copy(pe_hbm.at[pl.ds(base, C)], pe_v)

        def start_gather(s):
            b, h = divmod(s, 2)
            p = s % 3
            pltpu.sync_copy(x_hbm.at[b, pl.ds(base + h * H, H)], idx_v[p])
            return pltpu.async_copy(table_hbm.at[idx_v[p]], rows_v[p], gsem[p])

        gdesc = {0: start_gather(0), 1: start_gather(1)}
        wdesc = {}
        for s in range(NSTEP):
            p = s % 3
            b, h = divmod(s, 2)
            gdesc.pop(s).wait()
            if s + 2 < NSTEP:
                if s - 1 in wdesc:
                    wdesc.pop(s - 1).wait()
                gdesc[s + 2] = start_gather(s + 2)

            def add_pe_row(r, carry, _h=h, _p=p):
                for j in range(nj):
                    plsc.addupdate(rows_v[_p].at[r, pl.ds(j * 16, 16)],
                                   pe_v[_h * H + r, pl.ds(j * 16, 16)])
                return carry

            lax.fori_loop(0, H, add_pe_row, 0)
            wdesc[s] = pltpu.async_copy(
                rows_v[p], out_hbm.at[b, pl.ds(base + h * H, H)], wsem[p])
        for s in sorted(wdesc):
            wdesc.pop(s).wait()

    return emb(x2d, table, pe)


def kernel(x, table, pe):
    return _embed_lookup(x.astype(jnp.int32), table, pe.astype(jnp.float32))


# speedup vs baseline: 1.1127x; 1.1103x over previous
import functools

import jax
import jax.numpy as jnp
from jax import lax
from jax.experimental import pallas as pl
from jax.experimental.pallas import tpu as pltpu
from jax.experimental.pallas import tpu_sc as plsc


def _sc_geometry():
    try:
        info = plsc.get_sparse_core_info()
        return info.num_cores, info.num_subcores
    except Exception:
        return 2, 16


def _embed_lookup(x2d, table, pe):
    B, S = x2d.shape
    V, D = table.shape
    NC, NS = _sc_geometry()
    NW = NC * NS
    C = S // NW

    mesh = plsc.VectorSubcoreMesh(core_axis_name="c", subcore_axis_name="s")

    @functools.partial(
        pl.kernel,
        mesh=mesh,
        out_type=jax.ShapeDtypeStruct((B, S, D), jnp.float32),
        scratch_types=[
            [pltpu.VMEM((C // 2,), jnp.int32)] * 3,
            [pltpu.VMEM((C // 2, D), jnp.float32)] * 3,
            pltpu.VMEM((C, D), jnp.float32),
            [pltpu.SemaphoreType.DMA] * 3,
            [pltpu.SemaphoreType.DMA] * 3,
        ],
    )
    def emb(x_hbm, table_hbm, pe_hbm, out_hbm, idx_v, rows_v, pe_v, gsem, wsem):
        wid = lax.axis_index("s") * NC + lax.axis_index("c")
        base = wid * C
        H = C // 2
        NSTEP = 2 * B
        nj = D // 16
        pltpu.sync_copy(pe_hbm.at[pl.ds(base, C)], pe_v)

        def start_gather(s):
            b, h = divmod(s, 2)
            p = s % 3
            pltpu.sync_copy(x_hbm.at[b, pl.ds(base + h * H, H)], idx_v[p])
            return pltpu.async_copy(table_hbm.at[idx_v[p]], rows_v[p], gsem[p])

        gdesc = {0: start_gather(0), 1: start_gather(1)}
        wdesc = {}
        for s in range(NSTEP):
            p = s % 3
            b, h = divmod(s, 2)
            gdesc.pop(s).wait()
            if s + 2 < NSTEP:
                if s - 1 in wdesc:
                    wdesc.pop(s - 1).wait()
                gdesc[s + 2] = start_gather(s + 2)

            def add_pe_row(r, _h=h, _p=p):
                for j in range(nj):
                    plsc.addupdate(rows_v[_p].at[r, pl.ds(j * 16, 16)],
                                   pe_v[_h * H + r, pl.ds(j * 16, 16)])

            plsc.parallel_loop(0, H, 1, unroll=2)(add_pe_row)
            wdesc[s] = pltpu.async_copy(
                rows_v[p], out_hbm.at[b, pl.ds(base + h * H, H)], wsem[p])
        for s in sorted(wdesc):
            wdesc.pop(s).wait()

    return emb(x2d, table, pe)


def kernel(x, table, pe):
    return _embed_lookup(x.astype(jnp.int32), table, pe.astype(jnp.float32))
